# baseline (speedup 1.0000x reference)
import jax
import jax.numpy as jnp
from jax.experimental import pallas as pl
from jax.experimental.pallas import tpu as pltpu

_INF = 100000000.0
_K = 13
_BN = 5000


def _alignment_block(logits, pbT, labels, gb):
    C = logits.shape[1]
    G = gb.shape[0]
    iota_c = jax.lax.broadcasted_iota(jnp.int32, (C, G), 0)
    onehot = (iota_c == jnp.broadcast_to(labels, (C, G))).astype(jnp.float32)
    glogT = jax.lax.dot_general(
        onehot, logits, (((0,), (1,)), ((), ())),
        precision=jax.lax.Precision.HIGHEST,
        preferred_element_type=jnp.float32)
    scoresT = jax.nn.sigmoid(glogT)

    px1 = pbT[0:1, :] - 0.5 * pbT[2:3, :]
    py1 = pbT[1:2, :] - 0.5 * pbT[3:4, :]
    px2 = pbT[0:1, :] + 0.5 * pbT[2:3, :]
    py2 = pbT[1:2, :] + 0.5 * pbT[3:4, :]
    gx1 = gb[:, 0:1] - 0.5 * gb[:, 2:3]
    gy1 = gb[:, 1:2] - 0.5 * gb[:, 3:4]
    gx2 = gb[:, 0:1] + 0.5 * gb[:, 2:3]
    gy2 = gb[:, 1:2] + 0.5 * gb[:, 3:4]

    area1 = (px2 - px1) * (py2 - py1)
    area2 = (gx2 - gx1) * (gy2 - gy1)
    ltx = jnp.maximum(px1, gx1)
    lty = jnp.maximum(py1, gy1)
    rbx = jnp.minimum(px2, gx2)
    rby = jnp.minimum(py2, gy2)
    wx = jnp.maximum(rbx - ltx, 0.0)
    wy = jnp.maximum(rby - lty, 0.0)
    inter = wx * wy
    union = area2 + area1 - inter
    iou = inter / union
    ex1 = jnp.minimum(px1, gx1)
    ey1 = jnp.minimum(py1, gy1)
    ex2 = jnp.maximum(px2, gx2)
    ey2 = jnp.maximum(py2, gy2)
    ewx = jnp.maximum(ex2 - ex1, 0.0)
    ewy = jnp.maximum(ey2 - ey1, 0.0)
    area_e = ewx * ewy
    ovT = -(iou - (area_e - union) / area_e)

    o2 = ovT * ovT
    o6 = (o2 * o2) * o2
    alignT = scoresT * o6
    return alignT, ovT


def _fused_kernel(logits_ref, pboxesR_ref, glabels_ref, gboxes_ref,
                  inds_ref, mets_ref, ascr_ref, oscr_ref, mtop_ref, thr_ref):
    p = pl.program_id(1)
    t = pl.program_id(2)
    G = gboxes_ref.shape[1]
    BN = pboxesR_ref.shape[3]

    nb = pl.num_programs(2)

    @pl.when(p == 0)
    def _phase_compute():
        alignT, ovT = _alignment_block(logits_ref[0], pboxesR_ref[0, 0],
                                       glabels_ref[0], gboxes_ref[0])
        ascr_ref[t] = alignT
        oscr_ref[t] = ovT

        @pl.when(t == 0)
        def _():
            mtop_ref[...] = jnp.full(mtop_ref.shape, -1.0, jnp.float32)

        m1, m2, m3, m4, m5 = (mtop_ref[0], mtop_ref[1], mtop_ref[2],
                              mtop_ref[3], mtop_ref[4])
        nfull = BN // 128
        for s in range(nfull + (1 if BN % 128 else 0)):
            lo = s * 128
            x = alignT[:, lo:lo + 128]
            if x.shape[1] < 128:
                x = jnp.concatenate(
                    [x, jnp.full((G, 128 - x.shape[1]), -1.0, jnp.float32)],
                    axis=1)
            c1 = jnp.minimum(m1, x); m1 = jnp.maximum(m1, x)
            c2 = jnp.minimum(m2, c1); m2 = jnp.maximum(m2, c1)
            c3 = jnp.minimum(m3, c2); m3 = jnp.maximum(m3, c2)
            c4 = jnp.minimum(m4, c3); m4 = jnp.maximum(m4, c3)
            m5 = jnp.maximum(m5, c4)
        mtop_ref[0], mtop_ref[1] = m1, m2
        mtop_ref[2], mtop_ref[3] = m3, m4
        mtop_ref[4] = m5

        @pl.when(t == nb - 1)
        def _finalize():
            red = jnp.concatenate([m1, m2, m3, m4], axis=1)
            m = jnp.max(red, axis=1, keepdims=True)
            for _ in range(_K - 1):
                m = jnp.max(jnp.where(red < m, red, -1.0),
                            axis=1, keepdims=True)
            tau_cand = m
            bad = jnp.max(m5, axis=1, keepdims=True) >= tau_cand
            thr_ref[...] = tau_cand

            @pl.when(jnp.any(bad))
            def _fallback():
                mm = jnp.full((G, 1), -1.0, jnp.float32)
                for tt in range(nb):
                    mm = jnp.maximum(
                        mm, jnp.max(ascr_ref[tt], axis=1, keepdims=True))
                for _ in range(_K - 1):
                    nxt = jnp.full((G, 1), -1.0, jnp.float32)
                    for tt in range(nb):
                        a = ascr_ref[tt]
                        nxt = jnp.maximum(
                            nxt, jnp.max(jnp.where(a < mm, a, -1.0),
                                         axis=1, keepdims=True))
                    mm = nxt
                thr_ref[...] = jnp.where(bad, mm, tau_cand)

    @pl.when(p == 1)
    def _phase_assign():
        alignT = ascr_ref[t]
        ovT = oscr_ref[t]
        thresh = thr_ref[...]

        maskT = (alignT >= thresh) & (alignT > 0.0)
        movT = jnp.where(maskT, ovT, -_INF)
        rmax = jnp.max(movT, axis=0, keepdims=True)
        pos = rmax != -_INF
        iota_g = jax.lax.broadcasted_iota(jnp.int32, (G, BN), 0)
        idx = jnp.min(jnp.where(movT == rmax, iota_g, G),
                      axis=0, keepdims=True)
        inds = jnp.where(pos, idx + 1, 0)
        met = jnp.max(jnp.where(iota_g == idx, alignT, 0.0),
                      axis=0, keepdims=True)
        met = jnp.where(pos, met, 0.0)

        inds_ref[0, 0] = inds.astype(jnp.int32)
        mets_ref[0, 0] = met


def kernel(pred_logits, pred_boxes, gt_labels, gt_boxes):
    bs, N, C = pred_logits.shape
    G = gt_labels.shape[1]
    nb = N // _BN
    glabels = gt_labels.astype(jnp.int32).reshape(bs, 1, G)
    pboxesR = jnp.swapaxes(pred_boxes.reshape(bs, nb, _BN, 4), 2, 3)

    nbm = nb - 1
    inds, mets = pl.pallas_call(
        _fused_kernel,
        grid=(bs, 2, nb),
        in_specs=[
            pl.BlockSpec((1, _BN, C),
                         lambda b, p, t: (b, t * (1 - p) + p * nbm, 0)),
            pl.BlockSpec((1, 1, 4, _BN),
                         lambda b, p, t: (b, t * (1 - p) + p * nbm, 0, 0)),
            pl.BlockSpec((1, 1, G), lambda b, p, t: (b, 0, 0)),
            pl.BlockSpec((1, G, 4), lambda b, p, t: (b, 0, 0)),
        ],
        out_specs=(
            pl.BlockSpec((1, 1, 1, _BN), lambda b, p, t: (b, t, 0, 0)),
            pl.BlockSpec((1, 1, 1, _BN), lambda b, p, t: (b, t, 0, 0)),
        ),
        out_shape=(
            jax.ShapeDtypeStruct((bs, nb, 1, _BN), jnp.int32),
            jax.ShapeDtypeStruct((bs, nb, 1, _BN), jnp.float32),
        ),
        scratch_shapes=[
            pltpu.VMEM((nb, G, _BN), jnp.float32),
            pltpu.VMEM((nb, G, _BN), jnp.float32),
            pltpu.VMEM((5, G, 128), jnp.float32),
            pltpu.VMEM((G, 1), jnp.float32),
        ],
        compiler_params=pltpu.CompilerParams(
            dimension_semantics=("parallel", "arbitrary", "arbitrary")),
    )(pred_logits, pboxesR, glabels, gt_boxes)
    return inds.reshape(bs, N), mets.reshape(bs, N)


# speedup vs baseline: 1.0011x; 1.0011x over previous
import jax
import jax.numpy as jnp
from jax.experimental import pallas as pl
from jax.experimental.pallas import tpu as pltpu

_INF = 100000000.0
_K = 13
_BN = 10000


def _alignment_block(logits, pbT, labels, gb):
    C = logits.shape[1]
    G = gb.shape[0]
    iota_c = jax.lax.broadcasted_iota(jnp.int32, (C, G), 0)
    onehot = (iota_c == jnp.broadcast_to(labels, (C, G))).astype(jnp.float32)
    glogT = jax.lax.dot_general(
        onehot, logits, (((0,), (1,)), ((), ())),
        precision=jax.lax.Precision.HIGHEST,
        preferred_element_type=jnp.float32)
    scoresT = jax.nn.sigmoid(glogT)

    px1 = pbT[0:1, :] - 0.5 * pbT[2:3, :]
    py1 = pbT[1:2, :] - 0.5 * pbT[3:4, :]
    px2 = pbT[0:1, :] + 0.5 * pbT[2:3, :]
    py2 = pbT[1:2, :] + 0.5 * pbT[3:4, :]
    gx1 = gb[:, 0:1] - 0.5 * gb[:, 2:3]
    gy1 = gb[:, 1:2] - 0.5 * gb[:, 3:4]
    gx2 = gb[:, 0:1] + 0.5 * gb[:, 2:3]
    gy2 = gb[:, 1:2] + 0.5 * gb[:, 3:4]

    area1 = (px2 - px1) * (py2 - py1)
    area2 = (gx2 - gx1) * (gy2 - gy1)
    ltx = jnp.maximum(px1, gx1)
    lty = jnp.maximum(py1, gy1)
    rbx = jnp.minimum(px2, gx2)
    rby = jnp.minimum(py2, gy2)
    wx = jnp.maximum(rbx - ltx, 0.0)
    wy = jnp.maximum(rby - lty, 0.0)
    inter = wx * wy
    union = area2 + area1 - inter
    iou = inter / union
    ex1 = jnp.minimum(px1, gx1)
    ey1 = jnp.minimum(py1, gy1)
    ex2 = jnp.maximum(px2, gx2)
    ey2 = jnp.maximum(py2, gy2)
    ewx = jnp.maximum(ex2 - ex1, 0.0)
    ewy = jnp.maximum(ey2 - ey1, 0.0)
    area_e = ewx * ewy
    ovT = -(iou - (area_e - union) / area_e)

    o2 = ovT * ovT
    o6 = (o2 * o2) * o2
    alignT = scoresT * o6
    return alignT, ovT


def _fused_kernel(logits_ref, pboxesR_ref, glabels_ref, gboxes_ref,
                  inds_ref, mets_ref, ascr_ref, oscr_ref, mtop_ref, thr_ref):
    p = pl.program_id(1)
    t = pl.program_id(2)
    G = gboxes_ref.shape[1]
    BN = pboxesR_ref.shape[3]

    nb = pl.num_programs(2)

    @pl.when(p == 0)
    def _phase_compute():
        alignT, ovT = _alignment_block(logits_ref[0], pboxesR_ref[0, 0],
                                       glabels_ref[0], gboxes_ref[0])
        ascr_ref[t] = alignT
        oscr_ref[t] = ovT

        @pl.when(t == 0)
        def _():
            mtop_ref[...] = jnp.full(mtop_ref.shape, -1.0, jnp.float32)

        m1, m2, m3, m4, m5 = (mtop_ref[0], mtop_ref[1], mtop_ref[2],
                              mtop_ref[3], mtop_ref[4])
        nfull = BN // 128
        for s in range(nfull + (1 if BN % 128 else 0)):
            lo = s * 128
            x = alignT[:, lo:lo + 128]
            if x.shape[1] < 128:
                x = jnp.concatenate(
                    [x, jnp.full((G, 128 - x.shape[1]), -1.0, jnp.float32)],
                    axis=1)
            c1 = jnp.minimum(m1, x); m1 = jnp.maximum(m1, x)
            c2 = jnp.minimum(m2, c1); m2 = jnp.maximum(m2, c1)
            c3 = jnp.minimum(m3, c2); m3 = jnp.maximum(m3, c2)
            c4 = jnp.minimum(m4, c3); m4 = jnp.maximum(m4, c3)
            m5 = jnp.maximum(m5, c4)
        mtop_ref[0], mtop_ref[1] = m1, m2
        mtop_ref[2], mtop_ref[3] = m3, m4
        mtop_ref[4] = m5

        @pl.when(t == nb - 1)
        def _finalize():
            red = jnp.concatenate([m1, m2, m3, m4], axis=1)
            m = jnp.max(red, axis=1, keepdims=True)
            for _ in range(_K - 1):
                m = jnp.max(jnp.where(red < m, red, -1.0),
                            axis=1, keepdims=True)
            tau_cand = m
            bad = jnp.max(m5, axis=1, keepdims=True) >= tau_cand
            thr_ref[...] = tau_cand

            @pl.when(jnp.any(bad))
            def _fallback():
                mm = jnp.full((G, 1), -1.0, jnp.float32)
                for tt in range(nb):
                    mm = jnp.maximum(
                        mm, jnp.max(ascr_ref[tt], axis=1, keepdims=True))
                for _ in range(_K - 1):
                    nxt = jnp.full((G, 1), -1.0, jnp.float32)
                    for tt in range(nb):
                        a = ascr_ref[tt]
                        nxt = jnp.maximum(
                            nxt, jnp.max(jnp.where(a < mm, a, -1.0),
                                         axis=1, keepdims=True))
                    mm = nxt
                thr_ref[...] = jnp.where(bad, mm, tau_cand)

    @pl.when(p == 1)
    def _phase_assign():
        alignT = ascr_ref[t]
        ovT = oscr_ref[t]
        thresh = thr_ref[...]

        maskT = (alignT >= thresh) & (alignT > 0.0)
        movT = jnp.where(maskT, ovT, -_INF)
        rmax = jnp.max(movT, axis=0, keepdims=True)
        pos = rmax != -_INF
        iota_g = jax.lax.broadcasted_iota(jnp.int32, (G, BN), 0)
        idx = jnp.min(jnp.where(movT == rmax, iota_g, G),
                      axis=0, keepdims=True)
        inds = jnp.where(pos, idx + 1, 0)
        met = jnp.max(jnp.where(iota_g == idx, alignT, 0.0),
                      axis=0, keepdims=True)
        met = jnp.where(pos, met, 0.0)

        inds_ref[0, 0] = inds.astype(jnp.int32)
        mets_ref[0, 0] = met


def kernel(pred_logits, pred_boxes, gt_labels, gt_boxes):
    bs, N, C = pred_logits.shape
    G = gt_labels.shape[1]
    nb = N // _BN
    glabels = gt_labels.astype(jnp.int32).reshape(bs, 1, G)
    pboxesR = jnp.swapaxes(pred_boxes.reshape(bs, nb, _BN, 4), 2, 3)

    nbm = nb - 1
    inds, mets = pl.pallas_call(
        _fused_kernel,
        grid=(bs, 2, nb),
        in_specs=[
            pl.BlockSpec((1, _BN, C),
                         lambda b, p, t: (b, t * (1 - p) + p * nbm, 0)),
            pl.BlockSpec((1, 1, 4, _BN),
                         lambda b, p, t: (b, t * (1 - p) + p * nbm, 0, 0)),
            pl.BlockSpec((1, 1, G), lambda b, p, t: (b, 0, 0)),
            pl.BlockSpec((1, G, 4), lambda b, p, t: (b, 0, 0)),
        ],
        out_specs=(
            pl.BlockSpec((1, 1, 1, _BN), lambda b, p, t: (b, t, 0, 0)),
            pl.BlockSpec((1, 1, 1, _BN), lambda b, p, t: (b, t, 0, 0)),
        ),
        out_shape=(
            jax.ShapeDtypeStruct((bs, nb, 1, _BN), jnp.int32),
            jax.ShapeDtypeStruct((bs, nb, 1, _BN), jnp.float32),
        ),
        scratch_shapes=[
            pltpu.VMEM((nb, G, _BN), jnp.float32),
            pltpu.VMEM((nb, G, _BN), jnp.float32),
            pltpu.VMEM((5, G, 128), jnp.float32),
            pltpu.VMEM((G, 1), jnp.float32),
        ],
        compiler_params=pltpu.CompilerParams(
            dimension_semantics=("parallel", "arbitrary", "arbitrary")),
    )(pred_logits, pboxesR, glabels, gt_boxes)
    return inds.reshape(bs, N), mets.reshape(bs, N)


# speedup vs baseline: 1.0105x; 1.0094x over previous
import jax
import jax.numpy as jnp
from jax.experimental import pallas as pl
from jax.experimental.pallas import tpu as pltpu

_INF = 100000000.0
_K = 13
_BN = 10000


def _alignment_block(logits, pbT, labels, gb):
    C = logits.shape[1]
    G = gb.shape[0]
    iota_c = jax.lax.broadcasted_iota(jnp.int32, (C, G), 0)
    onehot = (iota_c == jnp.broadcast_to(labels, (C, G))).astype(jnp.float32)
    glogT = jax.lax.dot_general(
        onehot, logits, (((0,), (1,)), ((), ())),
        precision=jax.lax.Precision.HIGHEST,
        preferred_element_type=jnp.float32)
    scoresT = jax.nn.sigmoid(glogT)

    px1 = pbT[0:1, :] - 0.5 * pbT[2:3, :]
    py1 = pbT[1:2, :] - 0.5 * pbT[3:4, :]
    px2 = pbT[0:1, :] + 0.5 * pbT[2:3, :]
    py2 = pbT[1:2, :] + 0.5 * pbT[3:4, :]
    gx1 = gb[:, 0:1] - 0.5 * gb[:, 2:3]
    gy1 = gb[:, 1:2] - 0.5 * gb[:, 3:4]
    gx2 = gb[:, 0:1] + 0.5 * gb[:, 2:3]
    gy2 = gb[:, 1:2] + 0.5 * gb[:, 3:4]

    area1 = (px2 - px1) * (py2 - py1)
    area2 = (gx2 - gx1) * (gy2 - gy1)
    ltx = jnp.maximum(px1, gx1)
    lty = jnp.maximum(py1, gy1)
    rbx = jnp.minimum(px2, gx2)
    rby = jnp.minimum(py2, gy2)
    wx = jnp.maximum(rbx - ltx, 0.0)
    wy = jnp.maximum(rby - lty, 0.0)
    inter = wx * wy
    union = area2 + area1 - inter
    iou = inter / union
    ex1 = jnp.minimum(px1, gx1)
    ey1 = jnp.minimum(py1, gy1)
    ex2 = jnp.maximum(px2, gx2)
    ey2 = jnp.maximum(py2, gy2)
    ewx = jnp.maximum(ex2 - ex1, 0.0)
    ewy = jnp.maximum(ey2 - ey1, 0.0)
    area_e = ewx * ewy
    ovT = -(iou - (area_e - union) / area_e)

    o2 = ovT * ovT
    o6 = (o2 * o2) * o2
    alignT = scoresT * o6
    return alignT, ovT


def _fused_kernel(logits_ref, pboxesR_ref, glabels_ref, gboxes_ref,
                  inds_ref, mets_ref, ascr_ref, oscr_ref, mtop_ref, thr_ref):
    p = pl.program_id(1)
    t = pl.program_id(2)
    G = gboxes_ref.shape[1]
    BN = pboxesR_ref.shape[3]

    nb = pl.num_programs(2)

    @pl.when(p == 0)
    def _phase_compute():
        alignT, ovT = _alignment_block(logits_ref[0], pboxesR_ref[0, 0],
                                       glabels_ref[0], gboxes_ref[0])
        ascr_ref[t] = alignT
        oscr_ref[t] = ovT

        @pl.when(t == 0)
        def _():
            mtop_ref[...] = jnp.full(mtop_ref.shape, -1.0, jnp.float32)

        m1, m2, m3, m4, m5 = (mtop_ref[0], mtop_ref[1], mtop_ref[2],
                              mtop_ref[3], mtop_ref[4])
        nfull = BN // 128
        for s in range(nfull + (1 if BN % 128 else 0)):
            lo = s * 128
            x = alignT[:, lo:lo + 128]
            if x.shape[1] < 128:
                x = jnp.concatenate(
                    [x, jnp.full((G, 128 - x.shape[1]), -1.0, jnp.float32)],
                    axis=1)
            c1 = jnp.minimum(m1, x); m1 = jnp.maximum(m1, x)
            c2 = jnp.minimum(m2, c1); m2 = jnp.maximum(m2, c1)
            c3 = jnp.minimum(m3, c2); m3 = jnp.maximum(m3, c2)
            c4 = jnp.minimum(m4, c3); m4 = jnp.maximum(m4, c3)
            m5 = jnp.maximum(m5, c4)
        mtop_ref[0], mtop_ref[1] = m1, m2
        mtop_ref[2], mtop_ref[3] = m3, m4
        mtop_ref[4] = m5

        @pl.when(t == nb - 1)
        def _finalize():
            red = jnp.concatenate([m1, m2, m3, m4], axis=1)
            m = jnp.max(red, axis=1, keepdims=True)
            for _ in range(_K - 1):
                m = jnp.max(jnp.where(red < m, red, -1.0),
                            axis=1, keepdims=True)
            tau_cand = m
            bad = jnp.max(m5, axis=1, keepdims=True) >= tau_cand
            thr_ref[...] = tau_cand

            @pl.when(jnp.any(bad))
            def _fallback():
                mm = jnp.full((G, 1), -1.0, jnp.float32)
                for tt in range(nb):
                    mm = jnp.maximum(
                        mm, jnp.max(ascr_ref[tt], axis=1, keepdims=True))
                for _ in range(_K - 1):
                    nxt = jnp.full((G, 1), -1.0, jnp.float32)
                    for tt in range(nb):
                        a = ascr_ref[tt]
                        nxt = jnp.maximum(
                            nxt, jnp.max(jnp.where(a < mm, a, -1.0),
                                         axis=1, keepdims=True))
                    mm = nxt
                thr_ref[...] = jnp.where(bad, mm, tau_cand)

    @pl.when(p == 1)
    def _phase_assign():
        alignT = ascr_ref[t]
        ovT = oscr_ref[t]
        thresh = thr_ref[...]

        maskT = alignT >= jnp.maximum(thresh, 1e-38)
        movT = jnp.where(maskT, ovT, -_INF)
        rmax = jnp.max(movT, axis=0, keepdims=True)
        pos = rmax != -_INF
        iota_g = jax.lax.broadcasted_iota(jnp.int32, (G, BN), 0)
        win = movT == rmax
        idx = jnp.min(jnp.where(win, iota_g, G), axis=0, keepdims=True)
        inds = jnp.where(pos, idx + 1, 0)
        met = jnp.max(jnp.where(win, alignT, 0.0), axis=0, keepdims=True)
        met = jnp.where(pos, met, 0.0)

        inds_ref[0, 0] = inds.astype(jnp.int32)
        mets_ref[0, 0] = met


def kernel(pred_logits, pred_boxes, gt_labels, gt_boxes):
    bs, N, C = pred_logits.shape
    G = gt_labels.shape[1]
    nb = N // _BN
    glabels = gt_labels.astype(jnp.int32).reshape(bs, 1, G)
    pboxesR = jnp.swapaxes(pred_boxes.reshape(bs, nb, _BN, 4), 2, 3)

    nbm = nb - 1
    inds, mets = pl.pallas_call(
        _fused_kernel,
        grid=(bs, 2, nb),
        in_specs=[
            pl.BlockSpec((1, _BN, C),
                         lambda b, p, t: (b, t * (1 - p) + p * nbm, 0)),
            pl.BlockSpec((1, 1, 4, _BN),
                         lambda b, p, t: (b, t * (1 - p) + p * nbm, 0, 0)),
            pl.BlockSpec((1, 1, G), lambda b, p, t: (b, 0, 0)),
            pl.BlockSpec((1, G, 4), lambda b, p, t: (b, 0, 0)),
        ],
        out_specs=(
            pl.BlockSpec((1, 1, 1, _BN), lambda b, p, t: (b, t, 0, 0)),
            pl.BlockSpec((1, 1, 1, _BN), lambda b, p, t: (b, t, 0, 0)),
        ),
        out_shape=(
            jax.ShapeDtypeStruct((bs, nb, 1, _BN), jnp.int32),
            jax.ShapeDtypeStruct((bs, nb, 1, _BN), jnp.float32),
        ),
        scratch_shapes=[
            pltpu.VMEM((nb, G, _BN), jnp.float32),
            pltpu.VMEM((nb, G, _BN), jnp.float32),
            pltpu.VMEM((5, G, 128), jnp.float32),
            pltpu.VMEM((G, 1), jnp.float32),
        ],
        compiler_params=pltpu.CompilerParams(
            dimension_semantics=("parallel", "arbitrary", "arbitrary")),
    )(pred_logits, pboxesR, glabels, gt_boxes)
    return inds.reshape(bs, N), mets.reshape(bs, N)


# speedup vs baseline: 1.0197x; 1.0091x over previous
import jax
import jax.numpy as jnp
from jax.experimental import pallas as pl
from jax.experimental.pallas import tpu as pltpu

_INF = 100000000.0
_K = 13
_BN = 10000


def _alignment_block(logits, pbT, labels, gb):
    C = logits.shape[1]
    G = gb.shape[0]
    iota_c = jax.lax.broadcasted_iota(jnp.int32, (C, G), 0)
    onehot = (iota_c == jnp.broadcast_to(labels, (C, G))).astype(jnp.float32)
    glogT = jax.lax.dot_general(
        onehot, logits, (((0,), (1,)), ((), ())),
        precision=jax.lax.Precision.HIGHEST,
        preferred_element_type=jnp.float32)
    scoresT = jax.nn.sigmoid(glogT)

    px1 = pbT[0:1, :] - 0.5 * pbT[2:3, :]
    py1 = pbT[1:2, :] - 0.5 * pbT[3:4, :]
    px2 = pbT[0:1, :] + 0.5 * pbT[2:3, :]
    py2 = pbT[1:2, :] + 0.5 * pbT[3:4, :]
    gx1 = gb[:, 0:1] - 0.5 * gb[:, 2:3]
    gy1 = gb[:, 1:2] - 0.5 * gb[:, 3:4]
    gx2 = gb[:, 0:1] + 0.5 * gb[:, 2:3]
    gy2 = gb[:, 1:2] + 0.5 * gb[:, 3:4]

    area1 = (px2 - px1) * (py2 - py1)
    area2 = (gx2 - gx1) * (gy2 - gy1)
    ltx = jnp.maximum(px1, gx1)
    lty = jnp.maximum(py1, gy1)
    rbx = jnp.minimum(px2, gx2)
    rby = jnp.minimum(py2, gy2)
    wx = jnp.maximum(rbx - ltx, 0.0)
    wy = jnp.maximum(rby - lty, 0.0)
    inter = wx * wy
    union = area2 + area1 - inter
    iou = inter / union
    ex1 = jnp.minimum(px1, gx1)
    ey1 = jnp.minimum(py1, gy1)
    ex2 = jnp.maximum(px2, gx2)
    ey2 = jnp.maximum(py2, gy2)
    ewx = jnp.maximum(ex2 - ex1, 0.0)
    ewy = jnp.maximum(ey2 - ey1, 0.0)
    area_e = ewx * ewy
    ovT = (area_e - union) / area_e - iou

    o2 = ovT * ovT
    o6 = (o2 * o2) * o2
    alignT = scoresT * o6
    return alignT, ovT


def _fused_kernel(logits_ref, pboxesR_ref, glabels_ref, gboxes_ref,
                  inds_ref, mets_ref, ascr_ref, oscr_ref, mtop_ref, thr_ref):
    p = pl.program_id(1)
    t = pl.program_id(2)
    G = gboxes_ref.shape[1]
    BN = pboxesR_ref.shape[3]

    nb = pl.num_programs(2)

    @pl.when(p == 0)
    def _phase_compute():
        alignT, ovT = _alignment_block(logits_ref[0], pboxesR_ref[0, 0],
                                       glabels_ref[0], gboxes_ref[0])
        ascr_ref[t] = alignT
        oscr_ref[t] = ovT

        @pl.when(t == 0)
        def _():
            mtop_ref[...] = jnp.full(mtop_ref.shape, -1.0, jnp.float32)

        m1, m2, m3, m4, m5 = (mtop_ref[0], mtop_ref[1], mtop_ref[2],
                              mtop_ref[3], mtop_ref[4])
        nfull = BN // 128
        for s in range(nfull + (1 if BN % 128 else 0)):
            lo = s * 128
            x = alignT[:, lo:lo + 128]
            if x.shape[1] < 128:
                x = jnp.concatenate(
                    [x, jnp.full((G, 128 - x.shape[1]), -1.0, jnp.float32)],
                    axis=1)
            c1 = jnp.minimum(m1, x); m1 = jnp.maximum(m1, x)
            c2 = jnp.minimum(m2, c1); m2 = jnp.maximum(m2, c1)
            c3 = jnp.minimum(m3, c2); m3 = jnp.maximum(m3, c2)
            c4 = jnp.minimum(m4, c3); m4 = jnp.maximum(m4, c3)
            m5 = jnp.maximum(m5, c4)
        mtop_ref[0], mtop_ref[1] = m1, m2
        mtop_ref[2], mtop_ref[3] = m3, m4
        mtop_ref[4] = m5

        @pl.when(t == nb - 1)
        def _finalize():
            red = jnp.concatenate([m1, m2, m3, m4], axis=1)
            m = jnp.max(red, axis=1, keepdims=True)
            for _ in range(_K - 1):
                m = jnp.max(jnp.where(red < m, red, -1.0),
                            axis=1, keepdims=True)
            tau_cand = m
            bad = jnp.max(m5, axis=1, keepdims=True) >= tau_cand
            thr_ref[...] = tau_cand

            @pl.when(jnp.any(bad))
            def _fallback():
                mm = jnp.full((G, 1), -1.0, jnp.float32)
                for tt in range(nb):
                    mm = jnp.maximum(
                        mm, jnp.max(ascr_ref[tt], axis=1, keepdims=True))
                for _ in range(_K - 1):
                    nxt = jnp.full((G, 1), -1.0, jnp.float32)
                    for tt in range(nb):
                        a = ascr_ref[tt]
                        nxt = jnp.maximum(
                            nxt, jnp.max(jnp.where(a < mm, a, -1.0),
                                         axis=1, keepdims=True))
                    mm = nxt
                thr_ref[...] = jnp.where(bad, mm, tau_cand)

    @pl.when(p == 1)
    def _phase_assign():
        alignT = ascr_ref[t]
        ovT = oscr_ref[t]
        thresh = thr_ref[...]

        maskT = alignT >= jnp.maximum(thresh, 1e-38)
        movT = jnp.where(maskT, ovT, -_INF)
        rmax = jnp.max(movT, axis=0, keepdims=True)
        pos = rmax != -_INF
        iota_g = jax.lax.broadcasted_iota(jnp.int32, (G, BN), 0)
        win = movT == rmax
        idx = jnp.min(jnp.where(win, iota_g, G), axis=0, keepdims=True)
        inds = jnp.where(pos, idx + 1, 0)
        met = jnp.max(jnp.where(win, alignT, 0.0), axis=0, keepdims=True)
        met = jnp.where(pos, met, 0.0)

        inds_ref[0, 0] = inds.astype(jnp.int32)
        mets_ref[0, 0] = met


def kernel(pred_logits, pred_boxes, gt_labels, gt_boxes):
    bs, N, C = pred_logits.shape
    G = gt_labels.shape[1]
    nb = N // _BN
    glabels = gt_labels.astype(jnp.int32).reshape(bs, 1, G)
    pboxesR = jnp.swapaxes(pred_boxes.reshape(bs, nb, _BN, 4), 2, 3)

    nbm = nb - 1
    inds, mets = pl.pallas_call(
        _fused_kernel,
        grid=(bs, 2, nb),
        in_specs=[
            pl.BlockSpec((1, _BN, C),
                         lambda b, p, t: (b, t * (1 - p) + p * nbm, 0)),
            pl.BlockSpec((1, 1, 4, _BN),
                         lambda b, p, t: (b, t * (1 - p) + p * nbm, 0, 0)),
            pl.BlockSpec((1, 1, G), lambda b, p, t: (b, 0, 0)),
            pl.BlockSpec((1, G, 4), lambda b, p, t: (b, 0, 0)),
        ],
        out_specs=(
            pl.BlockSpec((1, 1, 1, _BN), lambda b, p, t: (b, t, 0, 0)),
            pl.BlockSpec((1, 1, 1, _BN), lambda b, p, t: (b, t, 0, 0)),
        ),
        out_shape=(
            jax.ShapeDtypeStruct((bs, nb, 1, _BN), jnp.int32),
            jax.ShapeDtypeStruct((bs, nb, 1, _BN), jnp.float32),
        ),
        scratch_shapes=[
            pltpu.VMEM((nb, G, _BN), jnp.float32),
            pltpu.VMEM((nb, G, _BN), jnp.float32),
            pltpu.VMEM((5, G, 128), jnp.float32),
            pltpu.VMEM((G, 1), jnp.float32),
        ],
        compiler_params=pltpu.CompilerParams(
            dimension_semantics=("parallel", "arbitrary", "arbitrary")),
    )(pred_logits, pboxesR, glabels, gt_boxes)
    return inds.reshape(bs, N), mets.reshape(bs, N)
